# baseline (speedup 1.0000x reference)
import functools

import jax
import jax.numpy as jnp
from jax import lax
from jax.experimental import pallas as pl
from jax.experimental.pallas import tpu as pltpu
from jax.experimental.pallas import tpu_sc as plsc

E = 100000
N_DIMS = 512
N_SM = 128
B = 64
NUM_SEG = 25000
NUM_TGT = 25000
MEM = 40000

BLK = 512
NB = 196
E_PAD = NB * BLK
SEG_W = 25088
NW = 32
PW = E_PAD // NW
GCH = 392
MBLK = 1024

f32 = jnp.float32
bf16 = jnp.bfloat16
_NEG_INF = float("-inf")

_SC_MESH = plsc.VectorSubcoreMesh(core_axis_name="c", subcore_axis_name="s")
_SC_PARAMS = pltpu.CompilerParams(needs_layout_passes=False,
                                  use_tc_tiling_on_sc=False)


def _wid():
    return lax.axis_index("s") * 2 + lax.axis_index("c")


def _prep_body(wp_ref, wl_ref, wr_ref, bp_ref, bl_ref, br_ref, qs_ref, qr_ref,
               qt_ref, wmem_ref, wrel_ref, ql_ref, qr_out_ref, wrel_bf_ref):
    wp = wp_ref[...]
    wl = wl_ref[...]
    wr = wr_ref[...]

    def dot(a, b):
        return jnp.dot(a, b, preferred_element_type=f32)

    wmem_ref[:, :N_SM] = dot(wp, wl[0:128])
    wmem_ref[:, N_SM:] = dot(wp, wr[0:128])
    wrel_ref[:, :N_SM] = dot(wp, wl[128:256])
    wrel_ref[:, N_SM:] = dot(wp, wr[128:256])
    qs = dot(qs_ref[...], wp)
    qr = dot(qr_ref[...], wp)
    qt = dot(qt_ref[...], wp)
    bp = bp_ref[...].reshape(1, N_SM)
    wl_sum = wl[0:128] + wl[128:256] + wl[256:384] + wl[384:512] + wl[512:640]
    wr_sum = wr[0:128] + wr[128:256] + wr[256:384] + wr[384:512] + wr[512:640]
    ql_ref[...] = (dot(qs, wl[256:384]) + dot(qr, wl[384:512])
                   + dot(qt, wl[512:640]) + dot(bp, wl_sum)
                   + bl_ref[...].reshape(1, N_SM))
    qr_out_ref[...] = (dot(qs, wr[256:384]) + dot(qr, wr[384:512])
                       + dot(qt, wr[512:640]) + dot(bp, wr_sum)
                       + br_ref[...].reshape(1, N_SM))
    wrel_bf_ref[...] = wrel_ref[...].astype(bf16)


def _prep(W_proj, W_left, W_right, b_proj, b_left, b_right, qs, qr, qt):
    return pl.pallas_call(
        _prep_body,
        out_shape=(
            jax.ShapeDtypeStruct((N_DIMS, 2 * N_SM), f32),
            jax.ShapeDtypeStruct((N_DIMS, 2 * N_SM), f32),
            jax.ShapeDtypeStruct((B, N_SM), f32),
            jax.ShapeDtypeStruct((B, N_SM), f32),
            jax.ShapeDtypeStruct((N_DIMS, 2 * N_SM), bf16),
        ),
    )(W_proj, W_left, W_right, b_proj, b_left, b_right, qs, qr, qt)


def _memproj_body(x_ref, w_ref, ml_ref, mr_ref):
    acc = jnp.dot(x_ref[...].astype(bf16), w_ref[...].astype(bf16),
                  preferred_element_type=f32)
    ml_ref[...] = acc[:, :N_SM].astype(bf16)
    mr_ref[...] = acc[:, N_SM:].astype(bf16)


def _memproj(mem, wmem):
    grid = pl.cdiv(MEM, MBLK)
    return pl.pallas_call(
        _memproj_body,
        grid=(grid,),
        in_specs=[
            pl.BlockSpec((MBLK, N_DIMS), lambda i: (i, 0)),
            pl.BlockSpec((N_DIMS, 2 * N_SM), lambda i: (0, 0)),
        ],
        out_specs=(
            pl.BlockSpec((MBLK, N_SM), lambda i: (i, 0)),
            pl.BlockSpec((MBLK, N_SM), lambda i: (i, 0)),
        ),
        out_shape=(
            jax.ShapeDtypeStruct((MEM, N_SM), bf16),
            jax.ShapeDtypeStruct((MEM, N_SM), bf16),
        ),
    )(mem, wmem)


def _gather_body(ml_hbm, mr_hbm, vi_hbm, vj_hbm, gl_hbm, gr_hbm,
                 idx_v, rows_v, sem):
    base = _wid() * PW

    def run(table, idxh, outh):
        def chunk(c, _):
            off = base + c * GCH
            pltpu.sync_copy(idxh.at[pl.ds(off, GCH)], idx_v)
            pltpu.async_copy(table.at[idx_v], rows_v, sem).wait()
            pltpu.sync_copy(rows_v, outh.at[pl.ds(off, GCH)])
            return 0

        lax.fori_loop(0, PW // GCH, chunk, 0)

    run(ml_hbm, vi_hbm, gl_hbm)
    run(mr_hbm, vj_hbm, gr_hbm)


_gather = functools.partial(
    pl.kernel,
    mesh=_SC_MESH,
    compiler_params=_SC_PARAMS,
    out_type=(
        jax.ShapeDtypeStruct((E_PAD, N_SM // 2), jnp.int32),
        jax.ShapeDtypeStruct((E_PAD, N_SM // 2), jnp.int32),
    ),
    scratch_types=[
        pltpu.VMEM((GCH,), jnp.int32),
        pltpu.VMEM((GCH, N_SM // 2), jnp.int32),
        pltpu.SemaphoreType.DMA,
    ],
)(_gather_body)


def _edge_body(rel_ref, gl_ref, gr_ref, na_ref, eg_ref, wrel_ref, ql_ref,
               qr_ref, wc_ref, bc_ref, att_ref, m_ref):
    i = pl.program_id(0)
    rlr = jnp.dot(rel_ref[...].astype(bf16), wrel_ref[...],
                  preferred_element_type=f32)
    eg = eg_ref[...].reshape(BLK, 1)
    na = na_ref[...].reshape(1, BLK)
    onehot = (eg == lax.broadcasted_iota(jnp.int32, (1, B), 1)).astype(f32)
    ql = jnp.dot(onehot, ql_ref[...], preferred_element_type=f32)
    qr = jnp.dot(onehot, qr_ref[...], preferred_element_type=f32)
    zl = gl_ref[...].astype(f32) + rlr[:, :N_SM] + ql
    zr = gr_ref[...].astype(f32) + rlr[:, N_SM:] + qr
    lh = jnp.where(zl >= 0, zl, 0.01 * zl)
    rh = jnp.where(zr >= 0, zr, 0.01 * zr)
    ch = jnp.dot(rh.astype(bf16), wc_ref[...].astype(bf16),
                 preferred_element_type=f32) + bc_ref[...].reshape(1, N_SM)
    logits = jnp.sum(lh * ch, axis=1).reshape(1, BLK)
    a = logits * na
    gidx = i * BLK + lax.broadcasted_iota(jnp.int32, (1, BLK), 1)
    a = jnp.where(gidx < E, a, _NEG_INF)
    att_ref[...] = a.reshape(1, 1, BLK)

    @pl.when(i == 0)
    def _():
        m_ref[...] = jnp.full((1, 16), _NEG_INF, f32)

    m_ref[...] = jnp.maximum(m_ref[...], jnp.max(a))


def _edge(rel_emb, gl, gr, na_p, eg_p, wrel_bf, ql, qr, wc, bc):
    return pl.pallas_call(
        _edge_body,
        grid=(NB,),
        in_specs=[
            pl.BlockSpec((BLK, N_DIMS), lambda i: (i, 0)),
            pl.BlockSpec((BLK, N_SM), lambda i: (i, 0)),
            pl.BlockSpec((BLK, N_SM), lambda i: (i, 0)),
            pl.BlockSpec((1, 1, BLK), lambda i: (i, 0, 0)),
            pl.BlockSpec((1, 1, BLK), lambda i: (i, 0, 0)),
            pl.BlockSpec((N_DIMS, 2 * N_SM), lambda i: (0, 0)),
            pl.BlockSpec((B, N_SM), lambda i: (0, 0)),
            pl.BlockSpec((B, N_SM), lambda i: (0, 0)),
            pl.BlockSpec((N_SM, N_SM), lambda i: (0, 0)),
            pl.BlockSpec((N_SM,), lambda i: (0,)),
        ],
        out_specs=(
            pl.BlockSpec((1, 1, BLK), lambda i: (i, 0, 0)),
            pl.BlockSpec((1, 16), lambda i: (0, 0)),
        ),
        out_shape=(
            jax.ShapeDtypeStruct((NB, 1, BLK), f32),
            jax.ShapeDtypeStruct((1, 16), f32),
        ),
    )(rel_emb, gl, gr, na_p, eg_p, wrel_bf, ql, qr, wc, bc)


def _seg_den_body(att_hbm, m_hbm, seg_hbm, ex_hbm, denp_hbm,
                  att_v, seg_v, ex_v, den_l, m_v):
    w = _wid()
    base = w * PW
    pltpu.sync_copy(att_hbm.at[pl.ds(base, PW)], att_v)
    pltpu.sync_copy(seg_hbm.at[pl.ds(base, PW)], seg_v)
    pltpu.sync_copy(m_hbm, m_v)
    m = m_v[...]

    def zero(k, _):
        den_l[pl.ds(k * 16, 16)] = jnp.zeros((16,), f32)
        return 0

    lax.fori_loop(0, SEG_W // 16, zero, 0)

    def body(k, _):
        sl = pl.ds(k * 16, 16)
        e = jnp.exp(att_v[sl] - m)
        ex_v[sl] = e
        plsc.addupdate_scatter(den_l, [seg_v[sl]], e)
        return 0

    lax.fori_loop(0, PW // 16, body, 0)
    pltpu.sync_copy(ex_v, ex_hbm.at[pl.ds(base, PW)])
    pltpu.sync_copy(den_l, denp_hbm.at[w])


_seg_den = functools.partial(
    pl.kernel,
    mesh=_SC_MESH,
    compiler_params=_SC_PARAMS,
    out_type=(
        jax.ShapeDtypeStruct((E_PAD,), f32),
        jax.ShapeDtypeStruct((NW, SEG_W), f32),
    ),
    scratch_types=[
        pltpu.VMEM((PW,), f32),
        pltpu.VMEM((PW,), jnp.int32),
        pltpu.VMEM((PW,), f32),
        pltpu.VMEM((SEG_W,), f32),
        pltpu.VMEM((16,), f32),
    ],
)(_seg_den_body)


def _merge_body(p_ref, o_ref):
    o_ref[...] = jnp.sum(p_ref[...], axis=0)


def _merge(parts, width):
    grid = width // BLK
    return pl.pallas_call(
        _merge_body,
        grid=(grid,),
        in_specs=[pl.BlockSpec((NW, BLK), lambda i: (0, i))],
        out_specs=pl.BlockSpec((BLK,), lambda i: (i,)),
        out_shape=jax.ShapeDtypeStruct((width,), f32),
    )(parts)


def _soft_body(ex_hbm, seg_hbm, den_hbm, eg_hbm, soft_hbm, egsp_hbm,
               ex_v, seg_v, eg_v, den_v, soft_v, eg_l):
    w = _wid()
    base = w * PW
    pltpu.sync_copy(ex_hbm.at[pl.ds(base, PW)], ex_v)
    pltpu.sync_copy(seg_hbm.at[pl.ds(base, PW)], seg_v)
    pltpu.sync_copy(eg_hbm.at[pl.ds(base, PW)], eg_v)
    pltpu.sync_copy(den_hbm, den_v)

    for j in range(B // 16):
        eg_l[pl.ds(j * 16, 16)] = jnp.zeros((16,), f32)

    def body(k, _):
        sl = pl.ds(k * 16, 16)
        d = plsc.load_gather(den_v, [seg_v[sl]])
        so = ex_v[sl] / jnp.maximum(d, jnp.float32(1e-30))
        soft_v[sl] = so
        plsc.addupdate_scatter(eg_l, [eg_v[sl]], so)
        return 0

    lax.fori_loop(0, PW // 16, body, 0)
    pltpu.sync_copy(soft_v, soft_hbm.at[pl.ds(base, PW)])
    pltpu.sync_copy(eg_l, egsp_hbm.at[w])


_soft = functools.partial(
    pl.kernel,
    mesh=_SC_MESH,
    compiler_params=_SC_PARAMS,
    out_type=(
        jax.ShapeDtypeStruct((E_PAD,), f32),
        jax.ShapeDtypeStruct((NW, B), f32),
    ),
    scratch_types=[
        pltpu.VMEM((PW,), f32),
        pltpu.VMEM((PW,), jnp.int32),
        pltpu.VMEM((PW,), jnp.int32),
        pltpu.VMEM((SEG_W,), f32),
        pltpu.VMEM((PW,), f32),
        pltpu.VMEM((B,), f32),
    ],
)(_soft_body)


def _aggregate_body(soft_hbm, eg_hbm, dst_hbm, egsp_hbm, outp_hbm,
                    soft_v, eg_v, dst_v, egp_v, inv_v, out_l):
    w = _wid()
    base = w * PW
    pltpu.sync_copy(soft_hbm.at[pl.ds(base, PW)], soft_v)
    pltpu.sync_copy(eg_hbm.at[pl.ds(base, PW)], eg_v)
    pltpu.sync_copy(dst_hbm.at[pl.ds(base, PW)], dst_v)
    pltpu.sync_copy(egsp_hbm, egp_v)

    for j in range(B // 16):
        inv_v[pl.ds(j * 16, 16)] = jnp.zeros((16,), f32)

    def acc(i, _):
        for j in range(B // 16):
            sl = pl.ds(j * 16, 16)
            inv_v[sl] = inv_v[sl] + egp_v[pl.ds(i * B + j * 16, 16)]
        return 0

    lax.fori_loop(0, NW, acc, 0)
    for j in range(B // 16):
        sl = pl.ds(j * 16, 16)
        inv_v[sl] = jnp.float32(1.0) / jnp.maximum(inv_v[sl], jnp.float32(1e-30))

    def zero(k, _):
        out_l[pl.ds(k * 16, 16)] = jnp.zeros((16,), f32)
        return 0

    lax.fori_loop(0, SEG_W // 16, zero, 0)

    def body(k, _):
        sl = pl.ds(k * 16, 16)
        iv = plsc.load_gather(inv_v, [eg_v[sl]])
        plsc.addupdate_scatter(out_l, [dst_v[sl]], soft_v[sl] * iv)
        return 0

    lax.fori_loop(0, PW // 16, body, 0)
    pltpu.sync_copy(out_l, outp_hbm.at[w])


_aggregate = functools.partial(
    pl.kernel,
    mesh=_SC_MESH,
    compiler_params=_SC_PARAMS,
    out_type=jax.ShapeDtypeStruct((NW, SEG_W), f32),
    scratch_types=[
        pltpu.VMEM((PW,), f32),
        pltpu.VMEM((PW,), jnp.int32),
        pltpu.VMEM((PW,), jnp.int32),
        pltpu.VMEM((NW * B,), f32),
        pltpu.VMEM((B,), f32),
        pltpu.VMEM((SEG_W,), f32),
    ],
)(_aggregate_body)


def kernel(node_attention, memorized_embedding, rel_emb, query_src_emb,
           query_rel_emb, query_time_emb, eg_idx, idx_vi, idx_vj, seg_src,
           dst_ids, W_proj, b_proj, W_left, b_left, W_right, b_right,
           W_center, b_center):
    pad = E_PAD - E
    na_p = jnp.pad(node_attention, (0, pad)).reshape(NB, 1, BLK)
    eg_f = jnp.pad(eg_idx, (0, pad))
    vi_p = jnp.pad(idx_vi, (0, pad))
    vj_p = jnp.pad(idx_vj, (0, pad))
    seg_p = jnp.pad(seg_src, (0, pad))
    dst_p = jnp.pad(dst_ids, (0, pad))

    wmem, wrel, ql, qr, wrel_bf = _prep(W_proj, W_left, W_right, b_proj,
                                        b_left, b_right, query_src_emb,
                                        query_rel_emb, query_time_emb)
    ml, mr = _memproj(memorized_embedding, wmem)
    as_i32 = lambda t: lax.bitcast_convert_type(
        t.reshape(MEM, N_SM // 2, 2), jnp.int32)
    gl32, gr32 = _gather(as_i32(ml), as_i32(mr), vi_p, vj_p)
    as_bf = lambda t: lax.bitcast_convert_type(t, bf16).reshape(E_PAD, N_SM)
    att, m = _edge(rel_emb, as_bf(gl32), as_bf(gr32), na_p,
                   eg_f.reshape(NB, 1, BLK), wrel_bf,
                   ql, qr, W_center, b_center)
    ex, den_p = _seg_den(att.reshape(E_PAD), m.reshape(16), seg_p)
    den = _merge(den_p, SEG_W)
    soft, egs_p = _soft(ex, seg_p, den, eg_f)
    out_p = _aggregate(soft, eg_f, dst_p, egs_p.reshape(NW * B))
    res = _merge(out_p, SEG_W)
    return res[:NUM_TGT]


# speedup vs baseline: 2.5411x; 2.5411x over previous
import functools

import jax
import jax.numpy as jnp
from jax import lax
from jax.experimental import pallas as pl
from jax.experimental.pallas import tpu as pltpu
from jax.experimental.pallas import tpu_sc as plsc

E = 100000
N_DIMS = 512
N_SM = 128
B = 64
NUM_SEG = 25000
NUM_TGT = 25000
MEM = 40000

BLK = 512
NB = 196
E_PAD = NB * BLK
SEG_W = 25088
NW = 32
PW = E_PAD // NW
GCH = 392
MBLK = 1024

f32 = jnp.float32
bf16 = jnp.bfloat16
_NEG_INF = float("-inf")

_SC_MESH = plsc.VectorSubcoreMesh(core_axis_name="c", subcore_axis_name="s")
_SC_PARAMS = pltpu.CompilerParams(needs_layout_passes=False)


def _wid():
    return lax.axis_index("s") * 2 + lax.axis_index("c")


def _prep_body(wp_ref, wl_ref, wr_ref, bp_ref, bl_ref, br_ref, qs_ref, qr_ref,
               qt_ref, wmem_ref, wrel_ref, ql_ref, qr_out_ref, wrel_bf_ref):
    wp = wp_ref[...]
    wl = wl_ref[...]
    wr = wr_ref[...]

    def dot(a, b):
        return jnp.dot(a, b, preferred_element_type=f32)

    wmem_ref[:, :N_SM] = dot(wp, wl[0:128])
    wmem_ref[:, N_SM:] = dot(wp, wr[0:128])
    wrel_ref[:, :N_SM] = dot(wp, wl[128:256])
    wrel_ref[:, N_SM:] = dot(wp, wr[128:256])
    qs = dot(qs_ref[...], wp)
    qr = dot(qr_ref[...], wp)
    qt = dot(qt_ref[...], wp)
    bp = bp_ref[...].reshape(1, N_SM)
    wl_sum = wl[0:128] + wl[128:256] + wl[256:384] + wl[384:512] + wl[512:640]
    wr_sum = wr[0:128] + wr[128:256] + wr[256:384] + wr[384:512] + wr[512:640]
    ql_ref[...] = (dot(qs, wl[256:384]) + dot(qr, wl[384:512])
                   + dot(qt, wl[512:640]) + dot(bp, wl_sum)
                   + bl_ref[...].reshape(1, N_SM))
    qr_out_ref[...] = (dot(qs, wr[256:384]) + dot(qr, wr[384:512])
                       + dot(qt, wr[512:640]) + dot(bp, wr_sum)
                       + br_ref[...].reshape(1, N_SM))
    wrel_bf_ref[...] = wrel_ref[...].astype(bf16)


def _prep(W_proj, W_left, W_right, b_proj, b_left, b_right, qs, qr, qt):
    return pl.pallas_call(
        _prep_body,
        out_shape=(
            jax.ShapeDtypeStruct((N_DIMS, 2 * N_SM), f32),
            jax.ShapeDtypeStruct((N_DIMS, 2 * N_SM), f32),
            jax.ShapeDtypeStruct((B, N_SM), f32),
            jax.ShapeDtypeStruct((B, N_SM), f32),
            jax.ShapeDtypeStruct((N_DIMS, 2 * N_SM), bf16),
        ),
    )(W_proj, W_left, W_right, b_proj, b_left, b_right, qs, qr, qt)


def _memproj_body(x_ref, w_ref, ml_ref, mr_ref):
    acc = jnp.dot(x_ref[...].astype(bf16), w_ref[...].astype(bf16),
                  preferred_element_type=f32)
    ml_ref[...] = acc[:, :N_SM]
    mr_ref[...] = acc[:, N_SM:]


def _memproj(mem, wmem):
    grid = pl.cdiv(MEM, MBLK)
    return pl.pallas_call(
        _memproj_body,
        grid=(grid,),
        in_specs=[
            pl.BlockSpec((MBLK, N_DIMS), lambda i: (i, 0)),
            pl.BlockSpec((N_DIMS, 2 * N_SM), lambda i: (0, 0)),
        ],
        out_specs=(
            pl.BlockSpec((MBLK, N_SM), lambda i: (i, 0)),
            pl.BlockSpec((MBLK, N_SM), lambda i: (i, 0)),
        ),
        out_shape=(
            jax.ShapeDtypeStruct((MEM, N_SM), f32),
            jax.ShapeDtypeStruct((MEM, N_SM), f32),
        ),
    )(mem, wmem)


def _gather_body(ml_hbm, mr_hbm, vi_hbm, vj_hbm, gl_hbm, gr_hbm,
                 idx_v, rows_v, sem):
    base = _wid() * PW

    def run(table, idxh, outh):
        def chunk(c, _):
            off = base + c * GCH
            pltpu.sync_copy(idxh.at[pl.ds(off, GCH)], idx_v)
            pltpu.async_copy(table.at[idx_v], rows_v, sem).wait()
            pltpu.sync_copy(rows_v, outh.at[pl.ds(off, GCH)])
            return 0

        lax.fori_loop(0, PW // GCH, chunk, 0)

    run(ml_hbm, vi_hbm, gl_hbm)
    run(mr_hbm, vj_hbm, gr_hbm)


_gather = functools.partial(
    pl.kernel,
    mesh=_SC_MESH,
    compiler_params=_SC_PARAMS,
    out_type=(
        jax.ShapeDtypeStruct((E_PAD, N_SM), f32),
        jax.ShapeDtypeStruct((E_PAD, N_SM), f32),
    ),
    scratch_types=[
        pltpu.VMEM((GCH,), jnp.int32),
        pltpu.VMEM((GCH, N_SM), f32),
        pltpu.SemaphoreType.DMA,
    ],
)(_gather_body)


def _edge_body(rel_ref, gl_ref, gr_ref, na_ref, eg_ref, wrel_ref, ql_ref,
               qr_ref, wc_ref, bc_ref, att_ref, m_ref):
    i = pl.program_id(0)
    rlr = jnp.dot(rel_ref[...].astype(bf16), wrel_ref[...],
                  preferred_element_type=f32)
    eg = eg_ref[...].reshape(BLK, 1)
    na = na_ref[...].reshape(1, BLK)
    onehot = (eg == lax.broadcasted_iota(jnp.int32, (1, B), 1)).astype(f32)
    ql = jnp.dot(onehot, ql_ref[...], preferred_element_type=f32)
    qr = jnp.dot(onehot, qr_ref[...], preferred_element_type=f32)
    zl = gl_ref[...] + rlr[:, :N_SM] + ql
    zr = gr_ref[...] + rlr[:, N_SM:] + qr
    lh = jnp.where(zl >= 0, zl, 0.01 * zl)
    rh = jnp.where(zr >= 0, zr, 0.01 * zr)
    ch = jnp.dot(rh.astype(bf16), wc_ref[...].astype(bf16),
                 preferred_element_type=f32) + bc_ref[...].reshape(1, N_SM)
    logits = jnp.sum(lh * ch, axis=1).reshape(1, BLK)
    a = logits * na
    gidx = i * BLK + lax.broadcasted_iota(jnp.int32, (1, BLK), 1)
    a = jnp.where(gidx < E, a, _NEG_INF)
    att_ref[...] = a.reshape(1, 1, BLK)

    @pl.when(i == 0)
    def _():
        m_ref[...] = jnp.full((1, 16), _NEG_INF, f32)

    m_ref[...] = jnp.maximum(m_ref[...], jnp.max(a))


def _edge(rel_emb, gl, gr, na_p, eg_p, wrel_bf, ql, qr, wc, bc):
    return pl.pallas_call(
        _edge_body,
        grid=(NB,),
        in_specs=[
            pl.BlockSpec((BLK, N_DIMS), lambda i: (i, 0)),
            pl.BlockSpec((BLK, N_SM), lambda i: (i, 0)),
            pl.BlockSpec((BLK, N_SM), lambda i: (i, 0)),
            pl.BlockSpec((1, 1, BLK), lambda i: (i, 0, 0)),
            pl.BlockSpec((1, 1, BLK), lambda i: (i, 0, 0)),
            pl.BlockSpec((N_DIMS, 2 * N_SM), lambda i: (0, 0)),
            pl.BlockSpec((B, N_SM), lambda i: (0, 0)),
            pl.BlockSpec((B, N_SM), lambda i: (0, 0)),
            pl.BlockSpec((N_SM, N_SM), lambda i: (0, 0)),
            pl.BlockSpec((N_SM,), lambda i: (0,)),
        ],
        out_specs=(
            pl.BlockSpec((1, 1, BLK), lambda i: (i, 0, 0)),
            pl.BlockSpec((1, 16), lambda i: (0, 0)),
        ),
        out_shape=(
            jax.ShapeDtypeStruct((NB, 1, BLK), f32),
            jax.ShapeDtypeStruct((1, 16), f32),
        ),
    )(rel_emb, gl, gr, na_p, eg_p, wrel_bf, ql, qr, wc, bc)


def _seg_den_body(att_hbm, m_hbm, seg_hbm, ex_hbm, denp_hbm,
                  att_v, seg_v, ex_v, den_l, m_v):
    w = _wid()
    base = w * PW
    pltpu.sync_copy(att_hbm.at[pl.ds(base, PW)], att_v)
    pltpu.sync_copy(seg_hbm.at[pl.ds(base, PW)], seg_v)
    pltpu.sync_copy(m_hbm, m_v)
    m = m_v[...]

    def zero(k, _):
        den_l[pl.ds(k * 16, 16)] = jnp.zeros((16,), f32)
        return 0

    lax.fori_loop(0, SEG_W // 16, zero, 0)

    def body(k, _):
        sl = pl.ds(k * 16, 16)
        e = jnp.exp(att_v[sl] - m)
        ex_v[sl] = e
        plsc.addupdate_scatter(den_l, [seg_v[sl]], e)
        return 0

    lax.fori_loop(0, PW // 16, body, 0)
    pltpu.sync_copy(ex_v, ex_hbm.at[pl.ds(base, PW)])
    pltpu.sync_copy(den_l, denp_hbm.at[w])


_seg_den = functools.partial(
    pl.kernel,
    mesh=_SC_MESH,
    compiler_params=_SC_PARAMS,
    out_type=(
        jax.ShapeDtypeStruct((E_PAD,), f32),
        jax.ShapeDtypeStruct((NW, SEG_W), f32),
    ),
    scratch_types=[
        pltpu.VMEM((PW,), f32),
        pltpu.VMEM((PW,), jnp.int32),
        pltpu.VMEM((PW,), f32),
        pltpu.VMEM((SEG_W,), f32),
        pltpu.VMEM((16,), f32),
    ],
)(_seg_den_body)


def _merge_body(p_ref, o_ref):
    o_ref[...] = jnp.sum(p_ref[...], axis=0)


def _merge(parts, width):
    grid = width // BLK
    return pl.pallas_call(
        _merge_body,
        grid=(grid,),
        in_specs=[pl.BlockSpec((NW, BLK), lambda i: (0, i))],
        out_specs=pl.BlockSpec((BLK,), lambda i: (i,)),
        out_shape=jax.ShapeDtypeStruct((width,), f32),
    )(parts)


def _soft_body(ex_hbm, seg_hbm, den_hbm, eg_hbm, soft_hbm, egsp_hbm,
               ex_v, seg_v, eg_v, den_v, soft_v, eg_l):
    w = _wid()
    base = w * PW
    pltpu.sync_copy(ex_hbm.at[pl.ds(base, PW)], ex_v)
    pltpu.sync_copy(seg_hbm.at[pl.ds(base, PW)], seg_v)
    pltpu.sync_copy(eg_hbm.at[pl.ds(base, PW)], eg_v)
    pltpu.sync_copy(den_hbm, den_v)

    for j in range(B // 16):
        eg_l[pl.ds(j * 16, 16)] = jnp.zeros((16,), f32)

    def body(k, _):
        sl = pl.ds(k * 16, 16)
        d = plsc.load_gather(den_v, [seg_v[sl]])
        so = ex_v[sl] / jnp.maximum(d, jnp.float32(1e-30))
        soft_v[sl] = so
        plsc.addupdate_scatter(eg_l, [eg_v[sl]], so)
        return 0

    lax.fori_loop(0, PW // 16, body, 0)
    pltpu.sync_copy(soft_v, soft_hbm.at[pl.ds(base, PW)])
    pltpu.sync_copy(eg_l, egsp_hbm.at[w])


_soft = functools.partial(
    pl.kernel,
    mesh=_SC_MESH,
    compiler_params=_SC_PARAMS,
    out_type=(
        jax.ShapeDtypeStruct((E_PAD,), f32),
        jax.ShapeDtypeStruct((NW, B), f32),
    ),
    scratch_types=[
        pltpu.VMEM((PW,), f32),
        pltpu.VMEM((PW,), jnp.int32),
        pltpu.VMEM((PW,), jnp.int32),
        pltpu.VMEM((SEG_W,), f32),
        pltpu.VMEM((PW,), f32),
        pltpu.VMEM((B,), f32),
    ],
)(_soft_body)


def _aggregate_body(soft_hbm, eg_hbm, dst_hbm, egsp_hbm, outp_hbm,
                    soft_v, eg_v, dst_v, egp_v, inv_v, out_l):
    w = _wid()
    base = w * PW
    pltpu.sync_copy(soft_hbm.at[pl.ds(base, PW)], soft_v)
    pltpu.sync_copy(eg_hbm.at[pl.ds(base, PW)], eg_v)
    pltpu.sync_copy(dst_hbm.at[pl.ds(base, PW)], dst_v)
    pltpu.sync_copy(egsp_hbm, egp_v)

    for j in range(B // 16):
        inv_v[pl.ds(j * 16, 16)] = jnp.zeros((16,), f32)

    def acc(i, _):
        for j in range(B // 16):
            sl = pl.ds(j * 16, 16)
            inv_v[sl] = inv_v[sl] + egp_v[pl.ds(i * B + j * 16, 16)]
        return 0

    lax.fori_loop(0, NW, acc, 0)
    for j in range(B // 16):
        sl = pl.ds(j * 16, 16)
        inv_v[sl] = jnp.float32(1.0) / jnp.maximum(inv_v[sl], jnp.float32(1e-30))

    def zero(k, _):
        out_l[pl.ds(k * 16, 16)] = jnp.zeros((16,), f32)
        return 0

    lax.fori_loop(0, SEG_W // 16, zero, 0)

    def body(k, _):
        sl = pl.ds(k * 16, 16)
        iv = plsc.load_gather(inv_v, [eg_v[sl]])
        plsc.addupdate_scatter(out_l, [dst_v[sl]], soft_v[sl] * iv)
        return 0

    lax.fori_loop(0, PW // 16, body, 0)
    pltpu.sync_copy(out_l, outp_hbm.at[w])


_aggregate = functools.partial(
    pl.kernel,
    mesh=_SC_MESH,
    compiler_params=_SC_PARAMS,
    out_type=jax.ShapeDtypeStruct((NW, SEG_W), f32),
    scratch_types=[
        pltpu.VMEM((PW,), f32),
        pltpu.VMEM((PW,), jnp.int32),
        pltpu.VMEM((PW,), jnp.int32),
        pltpu.VMEM((NW * B,), f32),
        pltpu.VMEM((B,), f32),
        pltpu.VMEM((SEG_W,), f32),
    ],
)(_aggregate_body)


def kernel(node_attention, memorized_embedding, rel_emb, query_src_emb,
           query_rel_emb, query_time_emb, eg_idx, idx_vi, idx_vj, seg_src,
           dst_ids, W_proj, b_proj, W_left, b_left, W_right, b_right,
           W_center, b_center):
    pad = E_PAD - E
    na_p = jnp.pad(node_attention, (0, pad)).reshape(NB, 1, BLK)
    eg_f = jnp.pad(eg_idx, (0, pad))
    vi_p = jnp.pad(idx_vi, (0, pad))
    vj_p = jnp.pad(idx_vj, (0, pad))
    seg_p = jnp.pad(seg_src, (0, pad))
    dst_p = jnp.pad(dst_ids, (0, pad))

    wmem, wrel, ql, qr, wrel_bf = _prep(W_proj, W_left, W_right, b_proj,
                                        b_left, b_right, query_src_emb,
                                        query_rel_emb, query_time_emb)
    ml, mr = _memproj(memorized_embedding, wmem)
    gl, gr = _gather(ml, mr, vi_p, vj_p)
    att, m = _edge(rel_emb, gl, gr, na_p, eg_f.reshape(NB, 1, BLK), wrel_bf,
                   ql, qr, W_center, b_center)
    ex, den_p = _seg_den(att.reshape(E_PAD), m.reshape(16), seg_p)
    den = _merge(den_p, SEG_W)
    soft, egs_p = _soft(ex, seg_p, den, eg_f)
    out_p = _aggregate(soft, eg_f, dst_p, egs_p.reshape(NW * B))
    res = _merge(out_p, SEG_W)
    return res[:NUM_TGT]


# speedup vs baseline: 3.6264x; 1.4271x over previous
import functools

import jax
import jax.numpy as jnp
from jax import lax
from jax.experimental import pallas as pl
from jax.experimental.pallas import tpu as pltpu
from jax.experimental.pallas import tpu_sc as plsc

E = 100000
N_DIMS = 512
N_SM = 128
B = 64
NUM_SEG = 25000
NUM_TGT = 25000
MEM = 40000

BLK = 1024
NB = 98
MERGE_BLK = 512
E_PAD = NB * BLK
SEG_W = 25088
NW = 32
PW = E_PAD // NW
GCH = 392
NBH = NB // 2
HALF = NBH * BLK
PW_H = HALF // NW
MBLK = 1024

f32 = jnp.float32
bf16 = jnp.bfloat16
_NEG_INF = float("-inf")

_SC_MESH = plsc.VectorSubcoreMesh(core_axis_name="c", subcore_axis_name="s")
_SC_PARAMS = pltpu.CompilerParams(needs_layout_passes=False)


def _wid():
    return lax.axis_index("s") * 2 + lax.axis_index("c")


def _prep_body(wp_ref, wl_ref, wr_ref, bp_ref, bl_ref, br_ref, qs_ref, qr_ref,
               qt_ref, wmem_ref, wrel_ref, ql_ref, qr_out_ref, wrel_bf_ref):
    wp = wp_ref[...]
    wl = wl_ref[...]
    wr = wr_ref[...]

    def dot(a, b):
        return jnp.dot(a, b, preferred_element_type=f32)

    wmem_ref[:, :N_SM] = dot(wp, wl[0:128])
    wmem_ref[:, N_SM:] = dot(wp, wr[0:128])
    wrel_ref[:, :N_SM] = dot(wp, wl[128:256])
    wrel_ref[:, N_SM:] = dot(wp, wr[128:256])
    qs = dot(qs_ref[...], wp)
    qr = dot(qr_ref[...], wp)
    qt = dot(qt_ref[...], wp)
    bp = bp_ref[...].reshape(1, N_SM)
    wl_sum = wl[0:128] + wl[128:256] + wl[256:384] + wl[384:512] + wl[512:640]
    wr_sum = wr[0:128] + wr[128:256] + wr[256:384] + wr[384:512] + wr[512:640]
    ql_ref[...] = (dot(qs, wl[256:384]) + dot(qr, wl[384:512])
                   + dot(qt, wl[512:640]) + dot(bp, wl_sum)
                   + bl_ref[...].reshape(1, N_SM))
    qr_out_ref[...] = (dot(qs, wr[256:384]) + dot(qr, wr[384:512])
                       + dot(qt, wr[512:640]) + dot(bp, wr_sum)
                       + br_ref[...].reshape(1, N_SM))
    wrel_bf_ref[...] = wrel_ref[...].astype(bf16)


def _prep(W_proj, W_left, W_right, b_proj, b_left, b_right, qs, qr, qt):
    return pl.pallas_call(
        _prep_body,
        out_shape=(
            jax.ShapeDtypeStruct((N_DIMS, 2 * N_SM), f32),
            jax.ShapeDtypeStruct((N_DIMS, 2 * N_SM), f32),
            jax.ShapeDtypeStruct((B, N_SM), f32),
            jax.ShapeDtypeStruct((B, N_SM), f32),
            jax.ShapeDtypeStruct((N_DIMS, 2 * N_SM), bf16),
        ),
    )(W_proj, W_left, W_right, b_proj, b_left, b_right, qs, qr, qt)


def _memproj_body(x_ref, w_ref, ml_ref, mr_ref):
    acc = jnp.dot(x_ref[...].astype(bf16), w_ref[...].astype(bf16),
                  preferred_element_type=f32)
    ml_ref[...] = acc[:, :N_SM]
    mr_ref[...] = acc[:, N_SM:]


def _memproj(mem, wmem):
    grid = pl.cdiv(MEM, MBLK)
    return pl.pallas_call(
        _memproj_body,
        grid=(grid,),
        in_specs=[
            pl.BlockSpec((MBLK, N_DIMS), lambda i: (i, 0)),
            pl.BlockSpec((N_DIMS, 2 * N_SM), lambda i: (0, 0)),
        ],
        out_specs=(
            pl.BlockSpec((MBLK, N_SM), lambda i: (i, 0)),
            pl.BlockSpec((MBLK, N_SM), lambda i: (i, 0)),
        ),
        out_shape=(
            jax.ShapeDtypeStruct((MEM, N_SM), f32),
            jax.ShapeDtypeStruct((MEM, N_SM), f32),
        ),
    )(mem, wmem)


def _gather_body(ml_hbm, mr_hbm, vi_hbm, vj_hbm, gl_hbm, gr_hbm,
                 idx_v, rows0, rows1, g0, g1, w0, w1):
    base = _wid() * PW_H
    rows = (rows0, rows1)
    gsem = (g0, g1)
    wsem = (w0, w1)
    nch = PW_H // GCH

    def run(table, idxh, outh):
        pltpu.sync_copy(idxh.at[pl.ds(base, PW_H)], idx_v)
        hg = [None] * nch
        hw = [None] * nch
        for c in range(nch):
            b = c % 2
            if c >= 2:
                hw[c - 2].wait()
            hg[c] = pltpu.async_copy(
                table.at[idx_v.at[pl.ds(c * GCH, GCH)]], rows[b], gsem[b])
            if c >= 1:
                hg[c - 1].wait()
                hw[c - 1] = pltpu.async_copy(
                    rows[(c - 1) % 2],
                    outh.at[pl.ds(base + (c - 1) * GCH, GCH)],
                    wsem[(c - 1) % 2])
        hg[nch - 1].wait()
        hw[nch - 1] = pltpu.async_copy(
            rows[(nch - 1) % 2],
            outh.at[pl.ds(base + (nch - 1) * GCH, GCH)],
            wsem[(nch - 1) % 2])
        hw[nch - 2].wait()
        hw[nch - 1].wait()

    run(ml_hbm, vi_hbm, gl_hbm)
    run(mr_hbm, vj_hbm, gr_hbm)


_gather = functools.partial(
    pl.kernel,
    mesh=_SC_MESH,
    compiler_params=_SC_PARAMS,
    out_type=(
        jax.ShapeDtypeStruct((HALF, N_SM), f32),
        jax.ShapeDtypeStruct((HALF, N_SM), f32),
    ),
    scratch_types=[
        pltpu.VMEM((PW_H,), jnp.int32),
        pltpu.VMEM((GCH, N_SM), f32),
        pltpu.VMEM((GCH, N_SM), f32),
        pltpu.SemaphoreType.DMA,
        pltpu.SemaphoreType.DMA,
        pltpu.SemaphoreType.DMA,
        pltpu.SemaphoreType.DMA,
    ],
)(_gather_body)


def _edge_body(h, has_dep, *refs):
    if has_dep:
        (rel_ref, gl_ref, gr_ref, na_ref, eg_ref, wrel_ref, ql_ref, qr_ref,
         wc_ref, bc_ref, _dep_ref, att_ref, m_ref) = refs
    else:
        (rel_ref, gl_ref, gr_ref, na_ref, eg_ref, wrel_ref, ql_ref, qr_ref,
         wc_ref, bc_ref, att_ref, m_ref) = refs
    i = pl.program_id(0)
    dn = (((0,), (1,)), ((), ()))
    dn0 = (((0,), (0,)), ((), ()))
    rlr = lax.dot_general(wrel_ref[...], rel_ref[...].astype(bf16), dn,
                          preferred_element_type=f32)
    eye = (lax.broadcasted_iota(jnp.int32, (N_SM, N_SM), 0)
           == lax.broadcasted_iota(jnp.int32, (N_SM, N_SM), 1)).astype(f32)
    glT = lax.dot_general(eye, gl_ref[...], dn,
                          preferred_element_type=f32)
    grT = lax.dot_general(eye, gr_ref[...], dn,
                          preferred_element_type=f32)
    eg = eg_ref[...].reshape(1, BLK)
    onehotT = (lax.broadcasted_iota(jnp.int32, (B, 1), 0) == eg).astype(f32)
    qlT = lax.dot_general(ql_ref[...], onehotT, dn0,
                          preferred_element_type=f32)
    qrT = lax.dot_general(qr_ref[...], onehotT, dn0,
                          preferred_element_type=f32)
    zl = glT + rlr[:N_SM, :] + qlT
    zr = grT + rlr[N_SM:, :] + qrT
    lh = jnp.where(zl >= 0, zl, 0.01 * zl)
    rh = jnp.where(zr >= 0, zr, 0.01 * zr)
    ch = lax.dot_general(wc_ref[...].astype(bf16), rh.astype(bf16), dn0,
                         preferred_element_type=f32) \
        + bc_ref[...].reshape(N_SM, 1)
    ones = jnp.full((1, N_SM), 1.0, f32)
    logits = lax.dot_general(ones, lh * ch, (((1,), (0,)), ((), ())),
                             preferred_element_type=f32)
    a = logits * na_ref[...].reshape(1, BLK)
    gidx = (i + h * NBH) * BLK + lax.broadcasted_iota(jnp.int32, (1, BLK), 1)
    a = jnp.where(gidx < E, a, _NEG_INF)
    att_ref[...] = a.reshape(1, 1, BLK)

    @pl.when(i == 0)
    def _():
        m_ref[...] = jnp.full((1, 16), _NEG_INF, f32)

    m_ref[...] = jnp.maximum(m_ref[...], jnp.max(a))


def _edge(h, rel_emb, gl, gr, na_p, eg_p, wrel_bf, ql, qr, wc, bc, dep=None):
    extra_specs = ([pl.BlockSpec((8, N_SM), lambda i: (0, 0))]
                   if dep is not None else [])
    extra_args = [dep] if dep is not None else []
    return pl.pallas_call(
        functools.partial(_edge_body, h, dep is not None),
        grid=(NBH,),
        in_specs=[
            pl.BlockSpec((BLK, N_DIMS), lambda i: (i + h * NBH, 0)),
            pl.BlockSpec((BLK, N_SM), lambda i: (i, 0)),
            pl.BlockSpec((BLK, N_SM), lambda i: (i, 0)),
            pl.BlockSpec((1, 1, BLK), lambda i: (i + h * NBH, 0, 0)),
            pl.BlockSpec((1, 1, BLK), lambda i: (i + h * NBH, 0, 0)),
            pl.BlockSpec((N_DIMS, 2 * N_SM), lambda i: (0, 0)),
            pl.BlockSpec((B, N_SM), lambda i: (0, 0)),
            pl.BlockSpec((B, N_SM), lambda i: (0, 0)),
            pl.BlockSpec((N_SM, N_SM), lambda i: (0, 0)),
            pl.BlockSpec((N_SM,), lambda i: (0,)),
        ] + extra_specs,
        out_specs=(
            pl.BlockSpec((1, 1, BLK), lambda i: (i, 0, 0)),
            pl.BlockSpec((1, 16), lambda i: (0, 0)),
        ),
        out_shape=(
            jax.ShapeDtypeStruct((NBH, 1, BLK), f32),
            jax.ShapeDtypeStruct((1, 16), f32),
        ),
    )(rel_emb, gl, gr, na_p, eg_p, wrel_bf, ql, qr, wc, bc, *extra_args)


def _seg_den_body(att_hbm, m_hbm, seg_hbm, ex_hbm, denp_hbm,
                  att_v, seg_v, ex_v, den_l, m_v):
    w = _wid()
    base = w * PW
    pltpu.sync_copy(att_hbm.at[pl.ds(base, PW)], att_v)
    pltpu.sync_copy(seg_hbm.at[pl.ds(base, PW)], seg_v)
    pltpu.sync_copy(m_hbm, m_v)
    m = jnp.maximum(m_v[pl.ds(0, 16)], m_v[pl.ds(16, 16)])

    def zero(k, _):
        den_l[pl.ds(k * 16, 16)] = jnp.zeros((16,), f32)
        return 0

    lax.fori_loop(0, SEG_W // 16, zero, 0)

    def body(k, _):
        sl = pl.ds(k * 16, 16)
        e = jnp.exp(att_v[sl] - m)
        ex_v[sl] = e
        plsc.addupdate_scatter(den_l, [seg_v[sl]], e)
        return 0

    lax.fori_loop(0, PW // 16, body, 0)
    pltpu.sync_copy(ex_v, ex_hbm.at[pl.ds(base, PW)])
    pltpu.sync_copy(den_l, denp_hbm.at[w])


_seg_den = functools.partial(
    pl.kernel,
    mesh=_SC_MESH,
    compiler_params=_SC_PARAMS,
    out_type=(
        jax.ShapeDtypeStruct((E_PAD,), f32),
        jax.ShapeDtypeStruct((NW, SEG_W), f32),
    ),
    scratch_types=[
        pltpu.VMEM((PW,), f32),
        pltpu.VMEM((PW,), jnp.int32),
        pltpu.VMEM((PW,), f32),
        pltpu.VMEM((SEG_W,), f32),
        pltpu.VMEM((32,), f32),
    ],
)(_seg_den_body)


def _merge_body(p_ref, o_ref):
    o_ref[...] = jnp.sum(p_ref[...], axis=0)


def _merge(parts, width):
    return pl.pallas_call(
        _merge_body,
        out_shape=jax.ShapeDtypeStruct((width,), f32),
    )(parts)


def _soft_body(ex_hbm, seg_hbm, den_hbm, eg_hbm, soft_hbm, egsp_hbm,
               ex_v, seg_v, eg_v, den_v, soft_v, eg_l):
    w = _wid()
    base = w * PW
    pltpu.sync_copy(ex_hbm.at[pl.ds(base, PW)], ex_v)
    pltpu.sync_copy(seg_hbm.at[pl.ds(base, PW)], seg_v)
    pltpu.sync_copy(eg_hbm.at[pl.ds(base, PW)], eg_v)
    pltpu.sync_copy(den_hbm, den_v)

    for j in range(B // 16):
        eg_l[pl.ds(j * 16, 16)] = jnp.zeros((16,), f32)

    def body(k, _):
        sl = pl.ds(k * 16, 16)
        d = plsc.load_gather(den_v, [seg_v[sl]])
        so = ex_v[sl] / jnp.maximum(d, jnp.float32(1e-30))
        soft_v[sl] = so
        plsc.addupdate_scatter(eg_l, [eg_v[sl]], so)
        return 0

    lax.fori_loop(0, PW // 16, body, 0)
    pltpu.sync_copy(soft_v, soft_hbm.at[pl.ds(base, PW)])
    pltpu.sync_copy(eg_l, egsp_hbm.at[w])


_soft = functools.partial(
    pl.kernel,
    mesh=_SC_MESH,
    compiler_params=_SC_PARAMS,
    out_type=(
        jax.ShapeDtypeStruct((E_PAD,), f32),
        jax.ShapeDtypeStruct((NW, B), f32),
    ),
    scratch_types=[
        pltpu.VMEM((PW,), f32),
        pltpu.VMEM((PW,), jnp.int32),
        pltpu.VMEM((PW,), jnp.int32),
        pltpu.VMEM((SEG_W,), f32),
        pltpu.VMEM((PW,), f32),
        pltpu.VMEM((B,), f32),
    ],
)(_soft_body)


def _aggregate_body(soft_hbm, eg_hbm, dst_hbm, egsp_hbm, outp_hbm,
                    soft_v, eg_v, dst_v, egp_v, inv_v, out_l):
    w = _wid()
    base = w * PW
    pltpu.sync_copy(soft_hbm.at[pl.ds(base, PW)], soft_v)
    pltpu.sync_copy(eg_hbm.at[pl.ds(base, PW)], eg_v)
    pltpu.sync_copy(dst_hbm.at[pl.ds(base, PW)], dst_v)
    pltpu.sync_copy(egsp_hbm, egp_v)

    for j in range(B // 16):
        inv_v[pl.ds(j * 16, 16)] = jnp.zeros((16,), f32)

    def acc(i, _):
        for j in range(B // 16):
            sl = pl.ds(j * 16, 16)
            inv_v[sl] = inv_v[sl] + egp_v[pl.ds(i * B + j * 16, 16)]
        return 0

    lax.fori_loop(0, NW, acc, 0)
    for j in range(B // 16):
        sl = pl.ds(j * 16, 16)
        inv_v[sl] = jnp.float32(1.0) / jnp.maximum(inv_v[sl], jnp.float32(1e-30))

    def zero(k, _):
        out_l[pl.ds(k * 16, 16)] = jnp.zeros((16,), f32)
        return 0

    lax.fori_loop(0, SEG_W // 16, zero, 0)

    def body(k, _):
        sl = pl.ds(k * 16, 16)
        iv = plsc.load_gather(inv_v, [eg_v[sl]])
        plsc.addupdate_scatter(out_l, [dst_v[sl]], soft_v[sl] * iv)
        return 0

    lax.fori_loop(0, PW // 16, body, 0)
    pltpu.sync_copy(out_l, outp_hbm.at[w])


_aggregate = functools.partial(
    pl.kernel,
    mesh=_SC_MESH,
    compiler_params=_SC_PARAMS,
    out_type=jax.ShapeDtypeStruct((NW, SEG_W), f32),
    scratch_types=[
        pltpu.VMEM((PW,), f32),
        pltpu.VMEM((PW,), jnp.int32),
        pltpu.VMEM((PW,), jnp.int32),
        pltpu.VMEM((NW * B,), f32),
        pltpu.VMEM((B,), f32),
        pltpu.VMEM((SEG_W,), f32),
    ],
)(_aggregate_body)


def kernel(node_attention, memorized_embedding, rel_emb, query_src_emb,
           query_rel_emb, query_time_emb, eg_idx, idx_vi, idx_vj, seg_src,
           dst_ids, W_proj, b_proj, W_left, b_left, W_right, b_right,
           W_center, b_center):
    pad = E_PAD - E
    na_p = jnp.pad(node_attention, (0, pad)).reshape(NB, 1, BLK)
    eg_f = jnp.pad(eg_idx, (0, pad))
    vi_p = jnp.pad(idx_vi, (0, pad))
    vj_p = jnp.pad(idx_vj, (0, pad))
    seg_p = jnp.pad(seg_src, (0, pad))
    dst_p = jnp.pad(dst_ids, (0, pad))

    wmem, wrel, ql, qr, wrel_bf = _prep(W_proj, W_left, W_right, b_proj,
                                        b_left, b_right, query_src_emb,
                                        query_rel_emb, query_time_emb)
    ml, mr = _memproj(memorized_embedding, wmem)
    eg_p = eg_f.reshape(NB, 1, BLK)
    gl0, gr0 = _gather(ml, mr, vi_p[:HALF], vj_p[:HALF])
    gl1, gr1 = _gather(ml, mr, vi_p[HALF:], vj_p[HALF:])
    att0, m0 = _edge(0, rel_emb, gl0, gr0, na_p, eg_p, wrel_bf,
                     ql, qr, W_center, b_center, dep=gl1)
    att1, m1 = _edge(1, rel_emb, gl1, gr1, na_p, eg_p, wrel_bf,
                     ql, qr, W_center, b_center)
    att = jnp.concatenate([att0.reshape(HALF), att1.reshape(HALF)])
    m2 = jnp.concatenate([m0, m1], axis=1).reshape(32)
    ex, den_p = _seg_den(att, m2, seg_p)
    den = _merge(den_p, SEG_W)
    soft, egs_p = _soft(ex, seg_p, den, eg_f)
    out_p = _aggregate(soft, eg_f, dst_p, egs_p.reshape(NW * B))
    res = _merge(out_p, SEG_W)
    return res[:NUM_TGT]


# speedup vs baseline: 3.9345x; 1.0849x over previous
import functools

import jax
import jax.numpy as jnp
from jax import lax
from jax.experimental import pallas as pl
from jax.experimental.pallas import tpu as pltpu
from jax.experimental.pallas import tpu_sc as plsc

E = 100000
N_DIMS = 512
N_SM = 128
B = 64
NUM_SEG = 25000
NUM_TGT = 25000
MEM = 40000

BLK = 1024
NB = 98
MERGE_BLK = 512
E_PAD = NB * BLK
SEG_W = 25088
NW = 32
PW = E_PAD // NW
GCH = 392
NBH = NB // 2
HALF = NBH * BLK
PW_H = HALF // NW
MBLK = 1024

f32 = jnp.float32
bf16 = jnp.bfloat16
_NEG_INF = float("-inf")

_SC_MESH = plsc.VectorSubcoreMesh(core_axis_name="c", subcore_axis_name="s")
_SC_PARAMS = pltpu.CompilerParams(needs_layout_passes=False)


def _wid():
    return lax.axis_index("s") * 2 + lax.axis_index("c")


def _prep_body(wp_ref, wl_ref, wr_ref, bp_ref, bl_ref, br_ref, qs_ref, qr_ref,
               qt_ref, wmem_ref, wrel_ref, ql_ref, qr_out_ref, wrel_bf_ref):
    wp = wp_ref[...]
    wl = wl_ref[...]
    wr = wr_ref[...]

    def dot(a, b):
        return jnp.dot(a, b, preferred_element_type=f32)

    wmem_ref[:, :N_SM] = dot(wp, wl[0:128])
    wmem_ref[:, N_SM:] = dot(wp, wr[0:128])
    wrel_ref[:, :N_SM] = dot(wp, wl[128:256])
    wrel_ref[:, N_SM:] = dot(wp, wr[128:256])
    qs = dot(qs_ref[...], wp)
    qr = dot(qr_ref[...], wp)
    qt = dot(qt_ref[...], wp)
    bp = bp_ref[...].reshape(1, N_SM)
    wl_sum = wl[0:128] + wl[128:256] + wl[256:384] + wl[384:512] + wl[512:640]
    wr_sum = wr[0:128] + wr[128:256] + wr[256:384] + wr[384:512] + wr[512:640]
    ql_ref[...] = (dot(qs, wl[256:384]) + dot(qr, wl[384:512])
                   + dot(qt, wl[512:640]) + dot(bp, wl_sum)
                   + bl_ref[...].reshape(1, N_SM))
    qr_out_ref[...] = (dot(qs, wr[256:384]) + dot(qr, wr[384:512])
                       + dot(qt, wr[512:640]) + dot(bp, wr_sum)
                       + br_ref[...].reshape(1, N_SM))
    wrel_bf_ref[...] = wrel_ref[...].astype(bf16)


def _prep(W_proj, W_left, W_right, b_proj, b_left, b_right, qs, qr, qt):
    return pl.pallas_call(
        _prep_body,
        out_shape=(
            jax.ShapeDtypeStruct((N_DIMS, 2 * N_SM), f32),
            jax.ShapeDtypeStruct((N_DIMS, 2 * N_SM), f32),
            jax.ShapeDtypeStruct((B, N_SM), f32),
            jax.ShapeDtypeStruct((B, N_SM), f32),
            jax.ShapeDtypeStruct((N_DIMS, 2 * N_SM), bf16),
        ),
    )(W_proj, W_left, W_right, b_proj, b_left, b_right, qs, qr, qt)


def _memproj_body(x_ref, w_ref, ml_ref, mr_ref):
    acc = jnp.dot(x_ref[...].astype(bf16), w_ref[...].astype(bf16),
                  preferred_element_type=f32)
    ml_ref[...] = acc[:, :N_SM]
    mr_ref[...] = acc[:, N_SM:]


def _memproj(mem, wmem):
    grid = pl.cdiv(MEM, MBLK)
    return pl.pallas_call(
        _memproj_body,
        grid=(grid,),
        in_specs=[
            pl.BlockSpec((MBLK, N_DIMS), lambda i: (i, 0)),
            pl.BlockSpec((N_DIMS, 2 * N_SM), lambda i: (0, 0)),
        ],
        out_specs=(
            pl.BlockSpec((MBLK, N_SM), lambda i: (i, 0)),
            pl.BlockSpec((MBLK, N_SM), lambda i: (i, 0)),
        ),
        out_shape=(
            jax.ShapeDtypeStruct((MEM, N_SM), f32),
            jax.ShapeDtypeStruct((MEM, N_SM), f32),
        ),
    )(mem, wmem)


def _gather_body(ml_hbm, mr_hbm, vi_hbm, vj_hbm, gl_hbm, gr_hbm,
                 idx_v, rows0, rows1, g0, g1, w0, w1):
    base = _wid() * PW_H
    rows = (rows0, rows1)
    gsem = (g0, g1)
    wsem = (w0, w1)
    nch = PW_H // GCH

    def run(table, idxh, outh):
        pltpu.sync_copy(idxh.at[pl.ds(base, PW_H)], idx_v)
        hg = [None] * nch
        hw = [None] * nch
        for c in range(nch):
            b = c % 2
            if c >= 2:
                hw[c - 2].wait()
            hg[c] = pltpu.async_copy(
                table.at[idx_v.at[pl.ds(c * GCH, GCH)]], rows[b], gsem[b])
            if c >= 1:
                hg[c - 1].wait()
                hw[c - 1] = pltpu.async_copy(
                    rows[(c - 1) % 2],
                    outh.at[pl.ds(base + (c - 1) * GCH, GCH)],
                    wsem[(c - 1) % 2])
        hg[nch - 1].wait()
        hw[nch - 1] = pltpu.async_copy(
            rows[(nch - 1) % 2],
            outh.at[pl.ds(base + (nch - 1) * GCH, GCH)],
            wsem[(nch - 1) % 2])
        hw[nch - 2].wait()
        hw[nch - 1].wait()

    run(ml_hbm, vi_hbm, gl_hbm)
    run(mr_hbm, vj_hbm, gr_hbm)


_gather = functools.partial(
    pl.kernel,
    mesh=_SC_MESH,
    compiler_params=_SC_PARAMS,
    out_type=(
        jax.ShapeDtypeStruct((HALF, N_SM), f32),
        jax.ShapeDtypeStruct((HALF, N_SM), f32),
    ),
    scratch_types=[
        pltpu.VMEM((PW_H,), jnp.int32),
        pltpu.VMEM((GCH, N_SM), f32),
        pltpu.VMEM((GCH, N_SM), f32),
        pltpu.SemaphoreType.DMA,
        pltpu.SemaphoreType.DMA,
        pltpu.SemaphoreType.DMA,
        pltpu.SemaphoreType.DMA,
    ],
)(_gather_body)


def _edge_body(h, rel_ref, gl_ref, gr_ref, na_ref, eg_ref, wrel_ref, ql_ref,
               qr_ref, wc_ref, bc_ref, att_ref, m_ref):
    i = pl.program_id(0)
    dn = (((0,), (1,)), ((), ()))
    dn0 = (((0,), (0,)), ((), ()))
    rlr = lax.dot_general(wrel_ref[...], rel_ref[...].astype(bf16), dn,
                          preferred_element_type=f32)
    eye = (lax.broadcasted_iota(jnp.int32, (N_SM, N_SM), 0)
           == lax.broadcasted_iota(jnp.int32, (N_SM, N_SM), 1)).astype(f32)
    glT = lax.dot_general(eye, gl_ref[...], dn,
                          preferred_element_type=f32)
    grT = lax.dot_general(eye, gr_ref[...], dn,
                          preferred_element_type=f32)
    eg = eg_ref[...].reshape(1, BLK)
    onehotT = (lax.broadcasted_iota(jnp.int32, (B, 1), 0) == eg).astype(f32)
    qlT = lax.dot_general(ql_ref[...], onehotT, dn0,
                          preferred_element_type=f32)
    qrT = lax.dot_general(qr_ref[...], onehotT, dn0,
                          preferred_element_type=f32)
    zl = glT + rlr[:N_SM, :] + qlT
    zr = grT + rlr[N_SM:, :] + qrT
    lh = jnp.where(zl >= 0, zl, 0.01 * zl)
    rh = jnp.where(zr >= 0, zr, 0.01 * zr)
    ch = lax.dot_general(wc_ref[...].astype(bf16), rh.astype(bf16), dn0,
                         preferred_element_type=f32) \
        + bc_ref[...].reshape(N_SM, 1)
    ones = jnp.full((1, N_SM), 1.0, f32)
    logits = lax.dot_general(ones, lh * ch, (((1,), (0,)), ((), ())),
                             preferred_element_type=f32)
    a = logits * na_ref[...].reshape(1, BLK)
    gidx = (i + h * NBH) * BLK + lax.broadcasted_iota(jnp.int32, (1, BLK), 1)
    a = jnp.where(gidx < E, a, _NEG_INF)
    att_ref[...] = a.reshape(1, 1, BLK)

    @pl.when(i == 0)
    def _():
        m_ref[...] = jnp.full((1, 16), _NEG_INF, f32)

    m_ref[...] = jnp.maximum(m_ref[...], jnp.max(a))


def _edge(h, rel_emb, gl, gr, na_p, eg_p, wrel_bf, ql, qr, wc, bc):
    return pl.pallas_call(
        functools.partial(_edge_body, h),
        grid=(NBH,),
        in_specs=[
            pl.BlockSpec((BLK, N_DIMS), lambda i: (i + h * NBH, 0)),
            pl.BlockSpec((BLK, N_SM), lambda i: (i, 0)),
            pl.BlockSpec((BLK, N_SM), lambda i: (i, 0)),
            pl.BlockSpec((1, 1, BLK), lambda i: (i + h * NBH, 0, 0)),
            pl.BlockSpec((1, 1, BLK), lambda i: (i + h * NBH, 0, 0)),
            pl.BlockSpec((N_DIMS, 2 * N_SM), lambda i: (0, 0)),
            pl.BlockSpec((B, N_SM), lambda i: (0, 0)),
            pl.BlockSpec((B, N_SM), lambda i: (0, 0)),
            pl.BlockSpec((N_SM, N_SM), lambda i: (0, 0)),
            pl.BlockSpec((N_SM,), lambda i: (0,)),
        ],
        out_specs=(
            pl.BlockSpec((1, 1, BLK), lambda i: (i, 0, 0)),
            pl.BlockSpec((1, 16), lambda i: (0, 0)),
        ),
        out_shape=(
            jax.ShapeDtypeStruct((NBH, 1, BLK), f32),
            jax.ShapeDtypeStruct((1, 16), f32),
        ),
    )(rel_emb, gl, gr, na_p, eg_p, wrel_bf, ql, qr, wc, bc)


def _seg_den_body(att_hbm, m_hbm, seg_hbm, ex_hbm, denp_hbm,
                  att_v, seg_v, ex_v, den_l, m_v):
    w = _wid()
    base = w * PW
    pltpu.sync_copy(att_hbm.at[pl.ds(base, PW)], att_v)
    pltpu.sync_copy(seg_hbm.at[pl.ds(base, PW)], seg_v)
    pltpu.sync_copy(m_hbm, m_v)
    m = jnp.maximum(m_v[pl.ds(0, 16)], m_v[pl.ds(16, 16)])

    def zero(k, _):
        for j in range(8):
            den_l[pl.ds((k * 8 + j) * 16, 16)] = jnp.zeros((16,), f32)
        return 0

    lax.fori_loop(0, SEG_W // 128, zero, 0)

    def body(k, _):
        for j in range(4):
            sl = pl.ds((k * 4 + j) * 16, 16)
            e = jnp.exp(att_v[sl] - m)
            ex_v[sl] = e
            plsc.addupdate_scatter(den_l, [seg_v[sl]], e)
        return 0

    lax.fori_loop(0, PW // 64, body, 0)
    pltpu.sync_copy(ex_v, ex_hbm.at[pl.ds(base, PW)])
    pltpu.sync_copy(den_l, denp_hbm.at[w])


_seg_den = functools.partial(
    pl.kernel,
    mesh=_SC_MESH,
    compiler_params=_SC_PARAMS,
    out_type=(
        jax.ShapeDtypeStruct((E_PAD,), f32),
        jax.ShapeDtypeStruct((NW, SEG_W), f32),
    ),
    scratch_types=[
        pltpu.VMEM((PW,), f32),
        pltpu.VMEM((PW,), jnp.int32),
        pltpu.VMEM((PW,), f32),
        pltpu.VMEM((SEG_W,), f32),
        pltpu.VMEM((32,), f32),
    ],
)(_seg_den_body)


def _merge_body(p_ref, o_ref):
    o_ref[...] = jnp.sum(p_ref[...], axis=0)


def _merge(parts, width):
    return pl.pallas_call(
        _merge_body,
        out_shape=jax.ShapeDtypeStruct((width,), f32),
    )(parts)


def _soft_body(ex_hbm, seg_hbm, den_hbm, eg_hbm, soft_hbm, egsp_hbm,
               ex_v, seg_v, eg_v, den_v, soft_v, eg_l):
    w = _wid()
    base = w * PW
    pltpu.sync_copy(ex_hbm.at[pl.ds(base, PW)], ex_v)
    pltpu.sync_copy(seg_hbm.at[pl.ds(base, PW)], seg_v)
    pltpu.sync_copy(eg_hbm.at[pl.ds(base, PW)], eg_v)
    pltpu.sync_copy(den_hbm, den_v)

    for j in range(B // 16):
        eg_l[pl.ds(j * 16, 16)] = jnp.zeros((16,), f32)

    def body(k, _):
        for j in range(4):
            sl = pl.ds((k * 4 + j) * 16, 16)
            d = plsc.load_gather(den_v, [seg_v[sl]])
            so = ex_v[sl] / jnp.maximum(d, jnp.float32(1e-30))
            soft_v[sl] = so
            plsc.addupdate_scatter(eg_l, [eg_v[sl]], so)
        return 0

    lax.fori_loop(0, PW // 64, body, 0)
    pltpu.sync_copy(soft_v, soft_hbm.at[pl.ds(base, PW)])
    pltpu.sync_copy(eg_l, egsp_hbm.at[w])


_soft = functools.partial(
    pl.kernel,
    mesh=_SC_MESH,
    compiler_params=_SC_PARAMS,
    out_type=(
        jax.ShapeDtypeStruct((E_PAD,), f32),
        jax.ShapeDtypeStruct((NW, B), f32),
    ),
    scratch_types=[
        pltpu.VMEM((PW,), f32),
        pltpu.VMEM((PW,), jnp.int32),
        pltpu.VMEM((PW,), jnp.int32),
        pltpu.VMEM((SEG_W,), f32),
        pltpu.VMEM((PW,), f32),
        pltpu.VMEM((B,), f32),
    ],
)(_soft_body)


def _aggregate_body(soft_hbm, eg_hbm, dst_hbm, egsp_hbm, outp_hbm,
                    soft_v, eg_v, dst_v, egp_v, inv_v, out_l):
    w = _wid()
    base = w * PW
    pltpu.sync_copy(soft_hbm.at[pl.ds(base, PW)], soft_v)
    pltpu.sync_copy(eg_hbm.at[pl.ds(base, PW)], eg_v)
    pltpu.sync_copy(dst_hbm.at[pl.ds(base, PW)], dst_v)
    pltpu.sync_copy(egsp_hbm, egp_v)

    for j in range(B // 16):
        inv_v[pl.ds(j * 16, 16)] = jnp.zeros((16,), f32)

    def acc(i, _):
        for j in range(B // 16):
            sl = pl.ds(j * 16, 16)
            inv_v[sl] = inv_v[sl] + egp_v[pl.ds(i * B + j * 16, 16)]
        return 0

    lax.fori_loop(0, NW, acc, 0)
    for j in range(B // 16):
        sl = pl.ds(j * 16, 16)
        inv_v[sl] = jnp.float32(1.0) / jnp.maximum(inv_v[sl], jnp.float32(1e-30))

    def zero(k, _):
        for j in range(8):
            out_l[pl.ds((k * 8 + j) * 16, 16)] = jnp.zeros((16,), f32)
        return 0

    lax.fori_loop(0, SEG_W // 128, zero, 0)

    def body(k, _):
        for j in range(4):
            sl = pl.ds((k * 4 + j) * 16, 16)
            iv = plsc.load_gather(inv_v, [eg_v[sl]])
            plsc.addupdate_scatter(out_l, [dst_v[sl]], soft_v[sl] * iv)
        return 0

    lax.fori_loop(0, PW // 64, body, 0)
    pltpu.sync_copy(out_l, outp_hbm.at[w])


_aggregate = functools.partial(
    pl.kernel,
    mesh=_SC_MESH,
    compiler_params=_SC_PARAMS,
    out_type=jax.ShapeDtypeStruct((NW, SEG_W), f32),
    scratch_types=[
        pltpu.VMEM((PW,), f32),
        pltpu.VMEM((PW,), jnp.int32),
        pltpu.VMEM((PW,), jnp.int32),
        pltpu.VMEM((NW * B,), f32),
        pltpu.VMEM((B,), f32),
        pltpu.VMEM((SEG_W,), f32),
    ],
)(_aggregate_body)


def kernel(node_attention, memorized_embedding, rel_emb, query_src_emb,
           query_rel_emb, query_time_emb, eg_idx, idx_vi, idx_vj, seg_src,
           dst_ids, W_proj, b_proj, W_left, b_left, W_right, b_right,
           W_center, b_center):
    pad = E_PAD - E
    na_p = jnp.pad(node_attention, (0, pad)).reshape(NB, 1, BLK)
    eg_f = jnp.pad(eg_idx, (0, pad))
    vi_p = jnp.pad(idx_vi, (0, pad))
    vj_p = jnp.pad(idx_vj, (0, pad))
    seg_p = jnp.pad(seg_src, (0, pad))
    dst_p = jnp.pad(dst_ids, (0, pad))

    wmem, wrel, ql, qr, wrel_bf = _prep(W_proj, W_left, W_right, b_proj,
                                        b_left, b_right, query_src_emb,
                                        query_rel_emb, query_time_emb)
    ml, mr = _memproj(memorized_embedding, wmem)
    eg_p = eg_f.reshape(NB, 1, BLK)
    gl0, gr0 = _gather(ml, mr, vi_p[:HALF], vj_p[:HALF])
    gl1, gr1 = _gather(ml, mr, vi_p[HALF:], vj_p[HALF:])
    att0, m0 = _edge(0, rel_emb, gl0, gr0, na_p, eg_p, wrel_bf,
                     ql, qr, W_center, b_center)
    att1, m1 = _edge(1, rel_emb, gl1, gr1, na_p, eg_p, wrel_bf,
                     ql, qr, W_center, b_center)
    att = jnp.concatenate([att0.reshape(HALF), att1.reshape(HALF)])
    m2 = jnp.concatenate([m0, m1], axis=1).reshape(32)
    ex, den_p = _seg_den(att, m2, seg_p)
    den = _merge(den_p, SEG_W)
    soft, egs_p = _soft(ex, seg_p, den, eg_f)
    out_p = _aggregate(soft, eg_f, dst_p, egs_p.reshape(NW * B))
    res = _merge(out_p, SEG_W)
    return res[:NUM_TGT]


# speedup vs baseline: 4.0825x; 1.0376x over previous
import functools

import jax
import jax.numpy as jnp
from jax import lax
from jax.experimental import pallas as pl
from jax.experimental.pallas import tpu as pltpu
from jax.experimental.pallas import tpu_sc as plsc

E = 100000
N_DIMS = 512
N_SM = 128
B = 64
NUM_SEG = 25000
NUM_TGT = 25000
MEM = 40000

BLK = 1024
NB = 98
MERGE_BLK = 512
E_PAD = NB * BLK
SEG_W = 25088
NW = 32
PW = E_PAD // NW
GCH = 392
NBH = NB // 2
HALF = NBH * BLK
PW_H = HALF // NW
MBLK = 2048

f32 = jnp.float32
bf16 = jnp.bfloat16
_NEG_INF = float("-inf")

_SC_MESH = plsc.VectorSubcoreMesh(core_axis_name="c", subcore_axis_name="s")
_SC_PARAMS = pltpu.CompilerParams(needs_layout_passes=False)


def _wid():
    return lax.axis_index("s") * 2 + lax.axis_index("c")


def _prep_body(wp_ref, wl_ref, wr_ref, bp_ref, bl_ref, br_ref, qs_ref, qr_ref,
               qt_ref, wmem_ref, wrel_ref, ql_ref, qr_out_ref, wrel_bf_ref):
    wp = wp_ref[...]
    wl = wl_ref[...]
    wr = wr_ref[...]

    def dot(a, b):
        return jnp.dot(a, b, preferred_element_type=f32)

    wmem_ref[:, :N_SM] = dot(wp, wl[0:128])
    wmem_ref[:, N_SM:] = dot(wp, wr[0:128])
    wrel_ref[:, :N_SM] = dot(wp, wl[128:256])
    wrel_ref[:, N_SM:] = dot(wp, wr[128:256])
    qs = dot(qs_ref[...], wp)
    qr = dot(qr_ref[...], wp)
    qt = dot(qt_ref[...], wp)
    bp = bp_ref[...].reshape(1, N_SM)
    wl_sum = wl[0:128] + wl[128:256] + wl[256:384] + wl[384:512] + wl[512:640]
    wr_sum = wr[0:128] + wr[128:256] + wr[256:384] + wr[384:512] + wr[512:640]
    ql_ref[...] = (dot(qs, wl[256:384]) + dot(qr, wl[384:512])
                   + dot(qt, wl[512:640]) + dot(bp, wl_sum)
                   + bl_ref[...].reshape(1, N_SM))
    qr_out_ref[...] = (dot(qs, wr[256:384]) + dot(qr, wr[384:512])
                       + dot(qt, wr[512:640]) + dot(bp, wr_sum)
                       + br_ref[...].reshape(1, N_SM))
    wrel_bf_ref[...] = wrel_ref[...].astype(bf16)


def _prep(W_proj, W_left, W_right, b_proj, b_left, b_right, qs, qr, qt):
    return pl.pallas_call(
        _prep_body,
        out_shape=(
            jax.ShapeDtypeStruct((N_DIMS, 2 * N_SM), f32),
            jax.ShapeDtypeStruct((N_DIMS, 2 * N_SM), f32),
            jax.ShapeDtypeStruct((B, N_SM), f32),
            jax.ShapeDtypeStruct((B, N_SM), f32),
            jax.ShapeDtypeStruct((N_DIMS, 2 * N_SM), bf16),
        ),
    )(W_proj, W_left, W_right, b_proj, b_left, b_right, qs, qr, qt)


def _memproj_body(x_ref, w_ref, ml_ref, mr_ref):
    acc = jnp.dot(x_ref[...].astype(bf16), w_ref[...].astype(bf16),
                  preferred_element_type=f32)
    ml_ref[...] = acc[:, :N_SM]
    mr_ref[...] = acc[:, N_SM:]


def _memproj(mem, wmem):
    grid = pl.cdiv(MEM, MBLK)
    return pl.pallas_call(
        _memproj_body,
        grid=(grid,),
        in_specs=[
            pl.BlockSpec((MBLK, N_DIMS), lambda i: (i, 0)),
            pl.BlockSpec((N_DIMS, 2 * N_SM), lambda i: (0, 0)),
        ],
        out_specs=(
            pl.BlockSpec((MBLK, N_SM), lambda i: (i, 0)),
            pl.BlockSpec((MBLK, N_SM), lambda i: (i, 0)),
        ),
        out_shape=(
            jax.ShapeDtypeStruct((MEM, N_SM), f32),
            jax.ShapeDtypeStruct((MEM, N_SM), f32),
        ),
    )(mem, wmem)


def _gather_body(ml_hbm, mr_hbm, vi_hbm, vj_hbm, gl_hbm, gr_hbm,
                 idx_v, rows0, rows1, g0, g1, w0, w1):
    base = _wid() * PW_H
    rows = (rows0, rows1)
    gsem = (g0, g1)
    wsem = (w0, w1)
    nch = PW_H // GCH

    def run(table, idxh, outh):
        pltpu.sync_copy(idxh.at[pl.ds(base, PW_H)], idx_v)
        hg = [None] * nch
        hw = [None] * nch
        for c in range(nch):
            b = c % 2
            if c >= 2:
                hw[c - 2].wait()
            hg[c] = pltpu.async_copy(
                table.at[idx_v.at[pl.ds(c * GCH, GCH)]], rows[b], gsem[b])
            if c >= 1:
                hg[c - 1].wait()
                hw[c - 1] = pltpu.async_copy(
                    rows[(c - 1) % 2],
                    outh.at[pl.ds(base + (c - 1) * GCH, GCH)],
                    wsem[(c - 1) % 2])
        hg[nch - 1].wait()
        hw[nch - 1] = pltpu.async_copy(
            rows[(nch - 1) % 2],
            outh.at[pl.ds(base + (nch - 1) * GCH, GCH)],
            wsem[(nch - 1) % 2])
        hw[nch - 2].wait()
        hw[nch - 1].wait()

    run(ml_hbm, vi_hbm, gl_hbm)
    run(mr_hbm, vj_hbm, gr_hbm)


_gather = functools.partial(
    pl.kernel,
    mesh=_SC_MESH,
    compiler_params=_SC_PARAMS,
    out_type=(
        jax.ShapeDtypeStruct((HALF, N_SM), f32),
        jax.ShapeDtypeStruct((HALF, N_SM), f32),
    ),
    scratch_types=[
        pltpu.VMEM((PW_H,), jnp.int32),
        pltpu.VMEM((GCH, N_SM), f32),
        pltpu.VMEM((GCH, N_SM), f32),
        pltpu.SemaphoreType.DMA,
        pltpu.SemaphoreType.DMA,
        pltpu.SemaphoreType.DMA,
        pltpu.SemaphoreType.DMA,
    ],
)(_gather_body)


def _edge_body(h, rel_ref, gl_ref, gr_ref, na_ref, eg_ref, wrel_ref, ql_ref,
               qr_ref, wc_ref, bc_ref, att_ref, m_ref):
    i = pl.program_id(0)
    dn = (((0,), (1,)), ((), ()))
    dn0 = (((0,), (0,)), ((), ()))
    rlr = lax.dot_general(wrel_ref[...], rel_ref[...].astype(bf16), dn,
                          preferred_element_type=f32)
    eye = (lax.broadcasted_iota(jnp.int32, (N_SM, N_SM), 0)
           == lax.broadcasted_iota(jnp.int32, (N_SM, N_SM), 1)).astype(f32)
    glT = lax.dot_general(eye, gl_ref[...], dn,
                          preferred_element_type=f32)
    grT = lax.dot_general(eye, gr_ref[...], dn,
                          preferred_element_type=f32)
    eg = eg_ref[...].reshape(1, BLK)
    onehotT = (lax.broadcasted_iota(jnp.int32, (B, 1), 0) == eg).astype(f32)
    qlT = lax.dot_general(ql_ref[...], onehotT, dn0,
                          preferred_element_type=f32)
    qrT = lax.dot_general(qr_ref[...], onehotT, dn0,
                          preferred_element_type=f32)
    zl = glT + rlr[:N_SM, :] + qlT
    zr = grT + rlr[N_SM:, :] + qrT
    lh = jnp.where(zl >= 0, zl, 0.01 * zl)
    rh = jnp.where(zr >= 0, zr, 0.01 * zr)
    ch = lax.dot_general(wc_ref[...].astype(bf16), rh.astype(bf16), dn0,
                         preferred_element_type=f32) \
        + bc_ref[...].reshape(N_SM, 1)
    ones = jnp.full((1, N_SM), 1.0, f32)
    logits = lax.dot_general(ones, lh * ch, (((1,), (0,)), ((), ())),
                             preferred_element_type=f32)
    a = logits * na_ref[...].reshape(1, BLK)
    gidx = (i + h * NBH) * BLK + lax.broadcasted_iota(jnp.int32, (1, BLK), 1)
    a = jnp.where(gidx < E, a, _NEG_INF)
    att_ref[...] = a.reshape(1, 1, BLK)

    @pl.when(i == 0)
    def _():
        m_ref[...] = jnp.full((1, 16), _NEG_INF, f32)

    m_ref[...] = jnp.maximum(m_ref[...], jnp.max(a))


def _edge(h, rel_emb, gl, gr, na_p, eg_p, wrel_bf, ql, qr, wc, bc):
    return pl.pallas_call(
        functools.partial(_edge_body, h),
        grid=(NBH,),
        in_specs=[
            pl.BlockSpec((BLK, N_DIMS), lambda i: (i + h * NBH, 0)),
            pl.BlockSpec((BLK, N_SM), lambda i: (i, 0)),
            pl.BlockSpec((BLK, N_SM), lambda i: (i, 0)),
            pl.BlockSpec((1, 1, BLK), lambda i: (i + h * NBH, 0, 0)),
            pl.BlockSpec((1, 1, BLK), lambda i: (i + h * NBH, 0, 0)),
            pl.BlockSpec((N_DIMS, 2 * N_SM), lambda i: (0, 0)),
            pl.BlockSpec((B, N_SM), lambda i: (0, 0)),
            pl.BlockSpec((B, N_SM), lambda i: (0, 0)),
            pl.BlockSpec((N_SM, N_SM), lambda i: (0, 0)),
            pl.BlockSpec((N_SM,), lambda i: (0,)),
        ],
        out_specs=(
            pl.BlockSpec((1, 1, BLK), lambda i: (i, 0, 0)),
            pl.BlockSpec((1, 16), lambda i: (0, 0)),
        ),
        out_shape=(
            jax.ShapeDtypeStruct((NBH, 1, BLK), f32),
            jax.ShapeDtypeStruct((1, 16), f32),
        ),
    )(rel_emb, gl, gr, na_p, eg_p, wrel_bf, ql, qr, wc, bc)


def _seg_den_body(att_hbm, m_hbm, seg_hbm, ex_hbm, denp_hbm,
                  att_v, seg_v, ex_v, den_l, m_v):
    w = _wid()
    base = w * PW
    pltpu.sync_copy(att_hbm.at[pl.ds(base, PW)], att_v)
    pltpu.sync_copy(seg_hbm.at[pl.ds(base, PW)], seg_v)
    pltpu.sync_copy(m_hbm, m_v)
    m = jnp.maximum(m_v[pl.ds(0, 16)], m_v[pl.ds(16, 16)])

    def zero(k, _):
        for j in range(8):
            den_l[pl.ds((k * 8 + j) * 16, 16)] = jnp.zeros((16,), f32)
        return 0

    lax.fori_loop(0, SEG_W // 128, zero, 0)

    def body(k, _):
        for j in range(4):
            sl = pl.ds((k * 4 + j) * 16, 16)
            e = jnp.exp(att_v[sl] - m)
            ex_v[sl] = e
            plsc.addupdate_scatter(den_l, [seg_v[sl]], e)
        return 0

    lax.fori_loop(0, PW // 64, body, 0)
    pltpu.sync_copy(ex_v, ex_hbm.at[pl.ds(base, PW)])
    pltpu.sync_copy(den_l, denp_hbm.at[w])


_seg_den = functools.partial(
    pl.kernel,
    mesh=_SC_MESH,
    compiler_params=_SC_PARAMS,
    out_type=(
        jax.ShapeDtypeStruct((E_PAD,), f32),
        jax.ShapeDtypeStruct((NW, SEG_W), f32),
    ),
    scratch_types=[
        pltpu.VMEM((PW,), f32),
        pltpu.VMEM((PW,), jnp.int32),
        pltpu.VMEM((PW,), f32),
        pltpu.VMEM((SEG_W,), f32),
        pltpu.VMEM((32,), f32),
    ],
)(_seg_den_body)


def _merge_body(p_ref, o_ref):
    o_ref[...] = jnp.sum(p_ref[...], axis=0)


def _merge(parts, width):
    return pl.pallas_call(
        _merge_body,
        out_shape=jax.ShapeDtypeStruct((width,), f32),
    )(parts)


def _soft_body(ex_hbm, seg_hbm, den_hbm, eg_hbm, soft_hbm, egsp_hbm,
               ex_v, seg_v, eg_v, den_v, soft_v, eg_l):
    w = _wid()
    base = w * PW
    pltpu.sync_copy(ex_hbm.at[pl.ds(base, PW)], ex_v)
    pltpu.sync_copy(seg_hbm.at[pl.ds(base, PW)], seg_v)
    pltpu.sync_copy(eg_hbm.at[pl.ds(base, PW)], eg_v)
    pltpu.sync_copy(den_hbm, den_v)

    for j in range(B // 16):
        eg_l[pl.ds(j * 16, 16)] = jnp.zeros((16,), f32)

    def body(k, _):
        for j in range(4):
            sl = pl.ds((k * 4 + j) * 16, 16)
            d = plsc.load_gather(den_v, [seg_v[sl]])
            so = ex_v[sl] / jnp.maximum(d, jnp.float32(1e-30))
            soft_v[sl] = so
            plsc.addupdate_scatter(eg_l, [eg_v[sl]], so)
        return 0

    lax.fori_loop(0, PW // 64, body, 0)
    pltpu.sync_copy(soft_v, soft_hbm.at[pl.ds(base, PW)])
    pltpu.sync_copy(eg_l, egsp_hbm.at[w])


_soft = functools.partial(
    pl.kernel,
    mesh=_SC_MESH,
    compiler_params=_SC_PARAMS,
    out_type=(
        jax.ShapeDtypeStruct((E_PAD,), f32),
        jax.ShapeDtypeStruct((NW, B), f32),
    ),
    scratch_types=[
        pltpu.VMEM((PW,), f32),
        pltpu.VMEM((PW,), jnp.int32),
        pltpu.VMEM((PW,), jnp.int32),
        pltpu.VMEM((SEG_W,), f32),
        pltpu.VMEM((PW,), f32),
        pltpu.VMEM((B,), f32),
    ],
)(_soft_body)


def _aggregate_body(soft_hbm, eg_hbm, dst_hbm, egsp_hbm, outp_hbm,
                    soft_v, eg_v, dst_v, egp_v, inv_v, out_l):
    w = _wid()
    base = w * PW
    pltpu.sync_copy(soft_hbm.at[pl.ds(base, PW)], soft_v)
    pltpu.sync_copy(eg_hbm.at[pl.ds(base, PW)], eg_v)
    pltpu.sync_copy(dst_hbm.at[pl.ds(base, PW)], dst_v)
    pltpu.sync_copy(egsp_hbm, egp_v)

    for j in range(B // 16):
        inv_v[pl.ds(j * 16, 16)] = jnp.zeros((16,), f32)

    def acc(i, _):
        for j in range(B // 16):
            sl = pl.ds(j * 16, 16)
            inv_v[sl] = inv_v[sl] + egp_v[pl.ds(i * B + j * 16, 16)]
        return 0

    lax.fori_loop(0, NW, acc, 0)
    for j in range(B // 16):
        sl = pl.ds(j * 16, 16)
        inv_v[sl] = jnp.float32(1.0) / jnp.maximum(inv_v[sl], jnp.float32(1e-30))

    def zero(k, _):
        for j in range(8):
            out_l[pl.ds((k * 8 + j) * 16, 16)] = jnp.zeros((16,), f32)
        return 0

    lax.fori_loop(0, SEG_W // 128, zero, 0)

    def body(k, _):
        for j in range(4):
            sl = pl.ds((k * 4 + j) * 16, 16)
            iv = plsc.load_gather(inv_v, [eg_v[sl]])
            plsc.addupdate_scatter(out_l, [dst_v[sl]], soft_v[sl] * iv)
        return 0

    lax.fori_loop(0, PW // 64, body, 0)
    pltpu.sync_copy(out_l, outp_hbm.at[w])


_aggregate = functools.partial(
    pl.kernel,
    mesh=_SC_MESH,
    compiler_params=_SC_PARAMS,
    out_type=jax.ShapeDtypeStruct((NW, SEG_W), f32),
    scratch_types=[
        pltpu.VMEM((PW,), f32),
        pltpu.VMEM((PW,), jnp.int32),
        pltpu.VMEM((PW,), jnp.int32),
        pltpu.VMEM((NW * B,), f32),
        pltpu.VMEM((B,), f32),
        pltpu.VMEM((SEG_W,), f32),
    ],
)(_aggregate_body)


def kernel(node_attention, memorized_embedding, rel_emb, query_src_emb,
           query_rel_emb, query_time_emb, eg_idx, idx_vi, idx_vj, seg_src,
           dst_ids, W_proj, b_proj, W_left, b_left, W_right, b_right,
           W_center, b_center):
    pad = E_PAD - E
    na_p = jnp.pad(node_attention, (0, pad)).reshape(NB, 1, BLK)
    eg_f = jnp.pad(eg_idx, (0, pad))
    vi_p = jnp.pad(idx_vi, (0, pad))
    vj_p = jnp.pad(idx_vj, (0, pad))
    seg_p = jnp.pad(seg_src, (0, pad))
    dst_p = jnp.pad(dst_ids, (0, pad))

    wmem, wrel, ql, qr, wrel_bf = _prep(W_proj, W_left, W_right, b_proj,
                                        b_left, b_right, query_src_emb,
                                        query_rel_emb, query_time_emb)
    ml, mr = _memproj(memorized_embedding, wmem)
    eg_p = eg_f.reshape(NB, 1, BLK)
    gl0, gr0 = _gather(ml, mr, vi_p[:HALF], vj_p[:HALF])
    gl1, gr1 = _gather(ml, mr, vi_p[HALF:], vj_p[HALF:])
    att0, m0 = _edge(0, rel_emb, gl0, gr0, na_p, eg_p, wrel_bf,
                     ql, qr, W_center, b_center)
    att1, m1 = _edge(1, rel_emb, gl1, gr1, na_p, eg_p, wrel_bf,
                     ql, qr, W_center, b_center)
    att = jnp.concatenate([att0.reshape(HALF), att1.reshape(HALF)])
    m2 = jnp.concatenate([m0, m1], axis=1).reshape(32)
    ex, den_p = _seg_den(att, m2, seg_p)
    den = _merge(den_p, SEG_W)
    soft, egs_p = _soft(ex, seg_p, den, eg_f)
    out_p = _aggregate(soft, eg_f, dst_p, egs_p.reshape(NW * B))
    res = _merge(out_p, SEG_W)
    return res[:NUM_TGT]
